# Initial kernel scaffold; baseline (speedup 1.0000x reference)
import jax
import jax.numpy as jnp
from jax.experimental import pallas as pl


def kernel(x, edge_index, W_l, b_l, W_r, bn_gamma, bn_beta):
    raise NotImplementedError("write your pallas kernel here")


# speedup vs baseline: 5.5837x; 5.5837x over previous
import functools

import jax
import jax.numpy as jnp
from jax import lax
from jax.experimental import pallas as pl
from jax.experimental.pallas import tpu as pltpu
from jax.experimental.pallas import tpu_sc as plsc

EPS = 1e-5
DP = 144


def _sc_segment_sum(xa, src, dst, zeros):
    n_nodes, dp = xa.shape
    n_edges = src.shape[0]
    nc, ns = 2, 16
    nw = nc * ns
    e_per_tile = n_edges // nw
    k = 80
    n_iter = e_per_tile // k
    assert e_per_tile * nw == n_edges and n_iter * k == e_per_tile
    rows_per_sub = n_nodes // ns
    assert rows_per_sub * ns == n_nodes

    mesh = plsc.VectorSubcoreMesh(core_axis_name="c", subcore_axis_name="s",
                                  num_cores=nc, num_subcores=ns)

    @functools.partial(
        pl.kernel,
        out_type=jax.ShapeDtypeStruct((nc, n_nodes, dp), jnp.float32),
        mesh=mesh,
        compiler_params=pltpu.CompilerParams(use_tc_tiling_on_sc=False),
        scratch_types=[
            pltpu.VMEM((k,), jnp.int32),
            pltpu.VMEM((k,), jnp.int32),
            pltpu.VMEM((k, dp), jnp.float32),
            pltpu.VMEM_SHARED((n_nodes, dp), jnp.float32),
            pltpu.SemaphoreType.DMA,
        ],
    )
    def seg_kernel(xa_hbm, src_hbm, dst_hbm, z_hbm, out_hbm,
                   src_v, dst_v, rows_v, agg_sh, sem):
        c = lax.axis_index("c")
        s = lax.axis_index("s")
        wid = s * nc + c
        pltpu.sync_copy(z_hbm.at[pl.ds(s * rows_per_sub, rows_per_sub)],
                        agg_sh.at[pl.ds(s * rows_per_sub, rows_per_sub)])
        plsc.subcore_barrier()
        base0 = wid * e_per_tile

        def body(i, carry):
            b = base0 + i * k
            pltpu.sync_copy(src_hbm.at[pl.ds(b, k)], src_v)
            pltpu.sync_copy(dst_hbm.at[pl.ds(b, k)], dst_v)
            pltpu.async_copy(xa_hbm.at[src_v], rows_v, sem).wait()
            pltpu.sync_copy(rows_v, agg_sh.at[dst_v], add=True)
            return carry

        lax.fori_loop(0, n_iter, body, 0)
        plsc.subcore_barrier()
        pltpu.sync_copy(agg_sh.at[pl.ds(s * rows_per_sub, rows_per_sub)],
                        out_hbm.at[c, pl.ds(s * rows_per_sub, rows_per_sub)])

    return seg_kernel(xa, src, dst, zeros)


def _tc_dense(parts, x, w_l, b_l, w_r, gamma, beta):
    n_nodes, d = x.shape

    def body(p_ref, x_ref, wl_ref, b_ref, wr_ref, g_ref, bt_ref, o_ref):
        a = p_ref[0] + p_ref[1]
        cnt = a[:, d:d + 1]
        mean = a[:, :d] / jnp.maximum(cnt, 1.0)
        h = lax.dot_general(mean, wl_ref[...], (((1,), (1,)), ((), ())),
                            preferred_element_type=jnp.float32)
        h = h + lax.dot_general(x_ref[...], wr_ref[...], (((1,), (1,)), ((), ())),
                                preferred_element_type=jnp.float32)
        h = h + b_ref[...]
        mu = jnp.mean(h, axis=0, keepdims=True)
        dev = h - mu
        var = jnp.mean(dev * dev, axis=0, keepdims=True)
        out = g_ref[...] * (dev * lax.rsqrt(var + EPS)) + bt_ref[...]
        o_ref[...] = jnp.maximum(out, 0.0)

    return pl.pallas_call(
        body,
        out_shape=jax.ShapeDtypeStruct((n_nodes, d), jnp.float32),
    )(parts, x, w_l, b_l.reshape(1, d), w_r, gamma.reshape(1, d),
      beta.reshape(1, d))


def kernel(x, edge_index, W_l, b_l, W_r, bn_gamma, bn_beta):
    n_nodes, d = x.shape
    src = edge_index[0]
    dst = edge_index[1]
    pad = jnp.zeros((n_nodes, DP - d), x.dtype).at[:, 0].set(1.0)
    xa = jnp.concatenate([x, pad], axis=1)
    zeros = jnp.zeros((n_nodes, DP), jnp.float32)
    parts = _sc_segment_sum(xa, src, dst, zeros)
    return _tc_dense(parts, x, W_l, b_l, W_r, bn_gamma, bn_beta)


# speedup vs baseline: 9.0086x; 1.6134x over previous
import functools

import jax
import jax.numpy as jnp
from jax import lax
from jax.experimental import pallas as pl
from jax.experimental.pallas import tpu as pltpu
from jax.experimental.pallas import tpu_sc as plsc

EPS = 1e-5
DP = 144


def _sc_segment_sum(xa, src3, dst3, zeros):
    n_nodes, dp = xa.shape
    nc, ns = 2, 16
    nw = nc * ns
    _, n_iter, k = src3.shape
    assert n_iter % 2 == 0 and n_iter >= 4
    rows_per_sub = n_nodes // ns
    assert rows_per_sub * ns == n_nodes

    mesh = plsc.VectorSubcoreMesh(core_axis_name="c", subcore_axis_name="s",
                                  num_cores=nc, num_subcores=ns)

    @functools.partial(
        pl.kernel,
        out_type=jax.ShapeDtypeStruct((nc, n_nodes, dp), jnp.float32),
        mesh=mesh,
        compiler_params=pltpu.CompilerParams(use_tc_tiling_on_sc=False),
        scratch_types=[
            pltpu.VMEM((n_iter, k), jnp.int32),
            pltpu.VMEM((n_iter, k), jnp.int32),
            pltpu.VMEM((k, dp), jnp.float32),
            pltpu.VMEM((k, dp), jnp.float32),
            pltpu.VMEM_SHARED((n_nodes, dp), jnp.float32),
            pltpu.SemaphoreType.DMA,
            pltpu.SemaphoreType.DMA,
        ],
    )
    def seg_kernel(xa_hbm, src_hbm, dst_hbm, z_hbm, out_hbm,
                   src_all, dst_all, rows_a, rows_b, agg_sh, sem_a, sem_b):
        c = lax.axis_index("c")
        s = lax.axis_index("s")
        wid = s * nc + c
        pltpu.sync_copy(z_hbm.at[pl.ds(s * rows_per_sub, rows_per_sub)],
                        agg_sh.at[pl.ds(s * rows_per_sub, rows_per_sub)])
        pltpu.sync_copy(src_hbm.at[wid], src_all)
        pltpu.sync_copy(dst_hbm.at[wid], dst_all)
        plsc.subcore_barrier()

        def g_start(i, rows, sem):
            pltpu.async_copy(xa_hbm.at[src_all.at[i]], rows, sem)

        def g_wait(rows, sem):
            pltpu.make_async_copy(xa_hbm.at[src_all.at[0]], rows, sem).wait()

        def s_add(i, rows):
            pltpu.sync_copy(rows, agg_sh.at[dst_all.at[i]], add=True)

        g_start(0, rows_a, sem_a)

        def body(j, carry):
            i0 = 2 * j
            g_start(i0 + 1, rows_b, sem_b)
            g_wait(rows_a, sem_a)
            s_add(i0, rows_a)
            g_start(i0 + 2, rows_a, sem_a)
            g_wait(rows_b, sem_b)
            s_add(i0 + 1, rows_b)
            return carry

        lax.fori_loop(0, n_iter // 2 - 1, body, 0)
        g_start(n_iter - 1, rows_b, sem_b)
        g_wait(rows_a, sem_a)
        s_add(n_iter - 2, rows_a)
        g_wait(rows_b, sem_b)
        s_add(n_iter - 1, rows_b)
        plsc.subcore_barrier()
        pltpu.sync_copy(agg_sh.at[pl.ds(s * rows_per_sub, rows_per_sub)],
                        out_hbm.at[c, pl.ds(s * rows_per_sub, rows_per_sub)])

    return seg_kernel(xa, src3, dst3, zeros)


def _tc_dense(parts, x, w_l, b_l, w_r, gamma, beta):
    n_nodes, d = x.shape

    def body(p_ref, x_ref, wl_ref, b_ref, wr_ref, g_ref, bt_ref, o_ref):
        a = p_ref[0] + p_ref[1]
        cnt = a[:, d:d + 1]
        mean = a[:, :d] / jnp.maximum(cnt, 1.0)
        h = lax.dot_general(mean, wl_ref[...], (((1,), (1,)), ((), ())),
                            preferred_element_type=jnp.float32)
        h = h + lax.dot_general(x_ref[...], wr_ref[...], (((1,), (1,)), ((), ())),
                                preferred_element_type=jnp.float32)
        h = h + b_ref[...]
        mu = jnp.mean(h, axis=0, keepdims=True)
        dev = h - mu
        var = jnp.mean(dev * dev, axis=0, keepdims=True)
        out = g_ref[...] * (dev * lax.rsqrt(var + EPS)) + bt_ref[...]
        o_ref[...] = jnp.maximum(out, 0.0)

    return pl.pallas_call(
        body,
        out_shape=jax.ShapeDtypeStruct((n_nodes, d), jnp.float32),
    )(parts, x, w_l, b_l.reshape(1, d), w_r, gamma.reshape(1, d),
      beta.reshape(1, d))


def kernel(x, edge_index, W_l, b_l, W_r, bn_gamma, bn_beta):
    n_nodes, d = x.shape
    n_edges = edge_index.shape[1]
    nw, k = 32, 40
    n_iter = n_edges // (nw * k)
    assert n_iter * nw * k == n_edges
    src3 = edge_index[0].reshape(nw, n_iter, k)
    dst3 = edge_index[1].reshape(nw, n_iter, k)
    pad = jnp.zeros((n_nodes, DP - d), x.dtype).at[:, 0].set(1.0)
    xa = jnp.concatenate([x, pad], axis=1)
    zeros = jnp.zeros((n_nodes, DP), jnp.float32)
    parts = _sc_segment_sum(xa, src3, dst3, zeros)
    return _tc_dense(parts, x, W_l, b_l, W_r, bn_gamma, bn_beta)
